# baseline (speedup 1.0000x reference)
import jax
import jax.numpy as jnp
from jax.experimental import pallas as pl

B, T, D, K = 16, 2048, 256, 1024
COMMITMENT_WEIGHT = 1.0
M = 512


def _vq_kernel(z_ref, z2_ref, mask_ref, cb_ref, c2_ref, q_ref, idx_ref,
               loss_ref):
    i = pl.program_id(0)

    z_blk = z_ref[...]
    cb = cb_ref[...]
    z2 = z2_ref[...]
    c2 = c2_ref[...]

    dots = jax.lax.dot_general(
        z_blk, cb,
        dimension_numbers=(((1,), (1,)), ((), ())),
        preferred_element_type=jnp.float32,
    )
    dist = z2 - 2.0 * dots + c2
    mins = jnp.min(dist, axis=1, keepdims=True)
    kiota = jax.lax.broadcasted_iota(jnp.int32, (M, K), 1)
    idx = jnp.min(jnp.where(dist == mins, kiota, K), axis=1).astype(jnp.int32)
    idx_ref[0, 0, :] = idx

    onehot = (kiota == idx[:, None]).astype(jnp.float32)
    q_blk = jax.lax.dot_general(
        onehot, cb,
        dimension_numbers=(((1,), (0,)), ((), ())),
        preferred_element_type=jnp.float32,
        precision=jax.lax.Precision.HIGHEST,
    )
    q_ref[...] = q_blk

    maskf = mask_ref[0, 0, :]
    diff = z_blk - q_blk
    part = jnp.sum(jnp.sum(diff * diff, axis=1) * maskf)

    @pl.when(i == 0)
    def _():
        loss_ref[...] = jnp.zeros_like(loss_ref)
    loss_ref[...] = loss_ref[...] + part


@jax.jit
def kernel(z, mask, codebook):
    BT = B * T
    nblk = BT // M
    zf = z.reshape(BT, D)
    z2 = jnp.sum(z * z, axis=-1, keepdims=True).reshape(BT, 1)
    c2 = jnp.sum(codebook * codebook, axis=-1).reshape(1, K)
    maskf = mask.astype(jnp.float32).reshape(nblk, 1, M)

    q, idx, loss_sum = pl.pallas_call(
        _vq_kernel,
        grid=(nblk,),
        in_specs=[
            pl.BlockSpec((M, D), lambda i: (i, 0)),
            pl.BlockSpec((M, 1), lambda i: (i, 0)),
            pl.BlockSpec((1, 1, M), lambda i: (i, 0, 0)),
            pl.BlockSpec((K, D), lambda i: (0, 0)),
            pl.BlockSpec((1, K), lambda i: (0, 0)),
        ],
        out_specs=[
            pl.BlockSpec((M, D), lambda i: (i, 0)),
            pl.BlockSpec((1, 1, M), lambda i: (i, 0, 0)),
            pl.BlockSpec((1, 1), lambda i: (0, 0)),
        ],
        out_shape=[
            jax.ShapeDtypeStruct((BT, D), jnp.float32),
            jax.ShapeDtypeStruct((nblk, 1, M), jnp.int32),
            jax.ShapeDtypeStruct((1, 1), jnp.float32),
        ],
    )(zf, z2, maskf, codebook, c2)

    quantized = q.reshape(B, T, D)
    indices = idx.reshape(B, T)
    denom = jnp.maximum(jnp.sum(mask.astype(jnp.float32)), 1.0) * D
    commit_loss = (loss_sum[0, 0] / denom) * COMMITMENT_WEIGHT

    quantized_st = z + jax.lax.stop_gradient(quantized - z)
    return quantized_st, indices, commit_loss


# speedup vs baseline: 1.2336x; 1.2336x over previous
import jax
import jax.numpy as jnp
from jax.experimental import pallas as pl

B, T, D, K = 16, 2048, 256, 1024
COMMITMENT_WEIGHT = 1.0
M = 512


def _vq_kernel(z_ref, z2_ref, mask_ref, cb_ref, c2_ref, cb0_ref, cb1_ref,
               cb2_ref, q_ref, idx_ref, loss_ref):
    i = pl.program_id(0)

    z_blk = z_ref[...]
    cb = cb_ref[...]
    z2 = z2_ref[...]
    c2 = c2_ref[...]

    dots = jax.lax.dot_general(
        z_blk, cb,
        dimension_numbers=(((1,), (1,)), ((), ())),
        preferred_element_type=jnp.float32,
    )
    dist = z2 - 2.0 * dots + c2
    mins = jnp.min(dist, axis=1, keepdims=True)
    kiota = jax.lax.broadcasted_iota(jnp.int32, (M, K), 1)
    idx = jnp.min(jnp.where(dist == mins, kiota, K), axis=1).astype(jnp.int32)
    idx_ref[0, 0, :] = idx

    onehot = (kiota == idx[:, None]).astype(jnp.bfloat16)
    dn = (((1,), (0,)), ((), ()))
    q_blk = jax.lax.dot_general(onehot, cb0_ref[...], dimension_numbers=dn,
                                preferred_element_type=jnp.float32)
    q_blk = q_blk + jax.lax.dot_general(onehot, cb1_ref[...],
                                        dimension_numbers=dn,
                                        preferred_element_type=jnp.float32)
    q_blk = q_blk + jax.lax.dot_general(onehot, cb2_ref[...],
                                        dimension_numbers=dn,
                                        preferred_element_type=jnp.float32)
    q_ref[...] = q_blk

    maskf = mask_ref[0, 0, :]
    diff = z_blk - q_blk
    part = jnp.sum(jnp.sum(diff * diff, axis=1) * maskf)

    @pl.when(i == 0)
    def _():
        loss_ref[...] = jnp.zeros_like(loss_ref)
    loss_ref[...] = loss_ref[...] + part


@jax.jit
def kernel(z, mask, codebook):
    BT = B * T
    nblk = BT // M
    zf = z.reshape(BT, D)
    z2 = jnp.sum(z * z, axis=-1, keepdims=True).reshape(BT, 1)
    c2 = jnp.sum(codebook * codebook, axis=-1).reshape(1, K)
    maskf = mask.astype(jnp.float32).reshape(nblk, 1, M)

    cb0 = codebook.astype(jnp.bfloat16)
    r1 = codebook - cb0.astype(jnp.float32)
    cb1 = r1.astype(jnp.bfloat16)
    cb2 = (r1 - cb1.astype(jnp.float32)).astype(jnp.bfloat16)

    q, idx, loss_sum = pl.pallas_call(
        _vq_kernel,
        grid=(nblk,),
        in_specs=[
            pl.BlockSpec((M, D), lambda i: (i, 0)),
            pl.BlockSpec((M, 1), lambda i: (i, 0)),
            pl.BlockSpec((1, 1, M), lambda i: (i, 0, 0)),
            pl.BlockSpec((K, D), lambda i: (0, 0)),
            pl.BlockSpec((1, K), lambda i: (0, 0)),
            pl.BlockSpec((K, D), lambda i: (0, 0)),
            pl.BlockSpec((K, D), lambda i: (0, 0)),
            pl.BlockSpec((K, D), lambda i: (0, 0)),
        ],
        out_specs=[
            pl.BlockSpec((M, D), lambda i: (i, 0)),
            pl.BlockSpec((1, 1, M), lambda i: (i, 0, 0)),
            pl.BlockSpec((1, 1), lambda i: (0, 0)),
        ],
        out_shape=[
            jax.ShapeDtypeStruct((BT, D), jnp.float32),
            jax.ShapeDtypeStruct((nblk, 1, M), jnp.int32),
            jax.ShapeDtypeStruct((1, 1), jnp.float32),
        ],
    )(zf, z2, maskf, codebook, c2, cb0, cb1, cb2)

    quantized = q.reshape(B, T, D)
    indices = idx.reshape(B, T)
    denom = jnp.maximum(jnp.sum(mask.astype(jnp.float32)), 1.0) * D
    commit_loss = (loss_sum[0, 0] / denom) * COMMITMENT_WEIGHT

    quantized_st = z + jax.lax.stop_gradient(quantized - z)
    return quantized_st, indices, commit_loss


# speedup vs baseline: 1.3235x; 1.0729x over previous
import functools

import jax
import jax.numpy as jnp
from jax import lax
from jax.experimental import pallas as pl
from jax.experimental.pallas import tpu as pltpu
from jax.experimental.pallas import tpu_sc as plsc

B, T, D, K = 16, 2048, 256, 1024
COMMITMENT_WEIGHT = 1.0
M = 512
BT = B * T
NBLK = BT // M

NW = 32
B_PER_W = BT // NW
CHUNK = 128
NCHUNK = B_PER_W // CHUNK


def _vq_tc_kernel(z_ref, z2_ref, mask_ref, cb_ref, c2_ref, idx_ref, loss_ref):
    i = pl.program_id(0)

    z_blk = z_ref[...]
    cb = cb_ref[...]
    z2 = z2_ref[...]
    c2 = c2_ref[...]

    dots = jax.lax.dot_general(
        z_blk, cb,
        dimension_numbers=(((1,), (1,)), ((), ())),
        preferred_element_type=jnp.float32,
    )
    dist = z2 - 2.0 * dots + c2
    mins = jnp.min(dist, axis=1, keepdims=True)
    kiota = jax.lax.broadcasted_iota(jnp.int32, (M, K), 1)
    idx = jnp.min(jnp.where(dist == mins, kiota, K), axis=1).astype(jnp.int32)
    idx_ref[0, 0, :] = idx

    maskf = mask_ref[0, 0, :]
    part = jnp.sum(mins[:, 0] * maskf)

    @pl.when(i == 0)
    def _():
        loss_ref[...] = jnp.zeros_like(loss_ref)
    loss_ref[...] = loss_ref[...] + part


def _gather_sc_body(cb_hbm, idx_hbm, out_hbm, idx_v, rows_v, sem):
    wid = lax.axis_index("s") * 2 + lax.axis_index("c")
    base = wid * B_PER_W
    for c in range(NCHUNK):
        pltpu.sync_copy(idx_hbm.at[pl.ds(base + c * CHUNK, CHUNK)], idx_v)
        pltpu.async_copy(cb_hbm.at[idx_v], rows_v, sem).wait()
        pltpu.sync_copy(rows_v, out_hbm.at[pl.ds(base + c * CHUNK, CHUNK)])


def _sc_gather(codebook, idx_flat):
    mesh = plsc.VectorSubcoreMesh(core_axis_name="c", subcore_axis_name="s")
    return pl.kernel(
        _gather_sc_body,
        out_type=jax.ShapeDtypeStruct((BT, D), jnp.float32),
        mesh=mesh,
        scratch_types=[
            pltpu.VMEM((CHUNK,), jnp.int32),
            pltpu.VMEM((CHUNK, D), jnp.float32),
            pltpu.SemaphoreType.DMA,
        ],
    )(codebook, idx_flat)


@jax.jit
def kernel(z, mask, codebook):
    zf = z.reshape(BT, D)
    z2 = jnp.sum(z * z, axis=-1, keepdims=True).reshape(BT, 1)
    c2 = jnp.sum(codebook * codebook, axis=-1).reshape(1, K)
    maskf = mask.astype(jnp.float32).reshape(NBLK, 1, M)

    idx, loss_sum = pl.pallas_call(
        _vq_tc_kernel,
        grid=(NBLK,),
        in_specs=[
            pl.BlockSpec((M, D), lambda i: (i, 0)),
            pl.BlockSpec((M, 1), lambda i: (i, 0)),
            pl.BlockSpec((1, 1, M), lambda i: (i, 0, 0)),
            pl.BlockSpec((K, D), lambda i: (0, 0)),
            pl.BlockSpec((1, K), lambda i: (0, 0)),
        ],
        out_specs=[
            pl.BlockSpec((1, 1, M), lambda i: (i, 0, 0)),
            pl.BlockSpec((1, 1), lambda i: (0, 0)),
        ],
        out_shape=[
            jax.ShapeDtypeStruct((NBLK, 1, M), jnp.int32),
            jax.ShapeDtypeStruct((1, 1), jnp.float32),
        ],
    )(zf, z2, maskf, codebook, c2)

    idx_flat = idx.reshape(BT)
    q = _sc_gather(codebook, idx_flat)

    quantized = q.reshape(B, T, D)
    indices = idx.reshape(B, T)
    denom = jnp.maximum(jnp.sum(mask.astype(jnp.float32)), 1.0) * D
    commit_loss = (loss_sum[0, 0] / denom) * COMMITMENT_WEIGHT

    quantized_st = z + jax.lax.stop_gradient(quantized - z)
    return quantized_st, indices, commit_loss
